# baseline (speedup 1.0000x reference)
import functools

import jax
import jax.numpy as jnp
from jax import lax
from jax.experimental import pallas as pl
from jax.experimental.pallas import tpu as pltpu
from jax.experimental.pallas import tpu_sc as plsc

N = 10000
E = 320000
H = 128
H2 = 64
EPS = 1e-07


_BN = 2000
_BE = 4000


def _enc_node_body(x_ref, w_ref, b_ref, out_ref):
    h = jnp.dot(x_ref[...], w_ref[...], preferred_element_type=jnp.float32)
    h = h + b_ref[...]
    out_ref[0] = h[:, :H2]
    out_ref[1] = h[:, H2:]


def _enc_node(x, W, b):
    return pl.pallas_call(
        _enc_node_body,
        grid=(N // _BN,),
        in_specs=[
            pl.BlockSpec((_BN, 3), lambda i: (i, 0)),
            pl.BlockSpec((3, H), lambda i: (0, 0)),
            pl.BlockSpec((1, H), lambda i: (0, 0)),
        ],
        out_specs=pl.BlockSpec((2, _BN, H2), lambda i: (0, i, 0)),
        out_shape=jax.ShapeDtypeStruct((2, N, H2), jnp.float32),
    )(x, W, b)


def _enc_edge_body(a_ref, w_ref, b_ref, out_ref):
    h = jnp.dot(a_ref[...], w_ref[...], preferred_element_type=jnp.float32)
    h = h + b_ref[...]
    out_ref[0] = h[:, :H2]
    out_ref[1] = h[:, H2:]


def _enc_edge(attr, W, b):
    return pl.pallas_call(
        _enc_edge_body,
        grid=(E // _BE,),
        in_specs=[
            pl.BlockSpec((_BE, 7), lambda i: (i, 0)),
            pl.BlockSpec((7, H), lambda i: (0, 0)),
            pl.BlockSpec((1, H), lambda i: (0, 0)),
        ],
        out_specs=pl.BlockSpec((2, _BE, H2), lambda i: (0, i, 0)),
        out_shape=jax.ShapeDtypeStruct((2, E, H2), jnp.float32),
    )(attr, W, b)


def _mlp_body(relu_out, h_ref, a_ref, w_ref, b_ref, g_ref, be_ref, out_ref):
    hp = jnp.concatenate([h_ref[0] + a_ref[0], h_ref[1] + a_ref[1]], axis=-1)
    t = jnp.dot(hp, w_ref[...], preferred_element_type=jnp.float32)
    t = t + b_ref[...]
    mu = jnp.mean(t, axis=-1, keepdims=True)
    var = jnp.mean((t - mu) * (t - mu), axis=-1, keepdims=True)
    y = (t - mu) / jnp.sqrt(var + 1e-5) * g_ref[...] + be_ref[...]
    if relu_out:
        y = jnp.maximum(y, 0.0)
        out_ref[0] = y[:, :H2]
        out_ref[1] = y[:, H2:]
    else:
        out_ref[...] = y


def _mlp(hs, aggr, Wc, bc, g, be, relu_out):
    if relu_out:
        out_spec = pl.BlockSpec((2, _BN, H2), lambda i: (0, i, 0))
        out_shape = jax.ShapeDtypeStruct((2, N, H2), jnp.float32)
    else:
        out_spec = pl.BlockSpec((_BN, H), lambda i: (i, 0))
        out_shape = jax.ShapeDtypeStruct((N, H), jnp.float32)
    return pl.pallas_call(
        functools.partial(_mlp_body, relu_out),
        grid=(N // _BN,),
        in_specs=[
            pl.BlockSpec((2, _BN, H2), lambda i: (0, i, 0)),
            pl.BlockSpec((2, _BN, H2), lambda i: (0, i, 0)),
            pl.BlockSpec((H, H), lambda i: (0, 0)),
            pl.BlockSpec((1, H), lambda i: (0, 0)),
            pl.BlockSpec((1, H), lambda i: (0, 0)),
            pl.BlockSpec((1, H), lambda i: (0, 0)),
        ],
        out_specs=out_spec,
        out_shape=out_shape,
    )(hs, aggr, Wc, bc, g, be)


_NSUB = 16
_C = 80
_NCH = E // _C
_CPT = _NCH // _NSUB
_NPT = N // _NSUB
_FC = 25

_mesh = plsc.VectorSubcoreMesh(core_axis_name="c", subcore_axis_name="s")


def _conv_body(h_hbm, emb_hbm, idx_hbm, out_hbm,
               idxb, hrows, erows, cbuf, SW, sem_h, sem_e, sem_s):
    cid = lax.axis_index("c")
    sid = lax.axis_index("s")
    cstart = sid * _CPT
    cnt = _CPT

    zero = jnp.zeros((16,), jnp.float32)

    def zbody(e, carry):
        for k in range(8):
            cbuf[0, e, pl.ds(k * 16, 16)] = zero
        return carry

    lax.fori_loop(0, _C, zbody, 0, unroll=False)
    nz_full, nz_tail = divmod(_NPT, _C)
    for j in range(nz_full):
        nb = sid * _NPT + j * _C
        pltpu.sync_copy(cbuf.at[0], SW.at[pl.ds(nb, _C)])
    if nz_tail:
        nb = sid * _NPT + nz_full * _C
        pltpu.sync_copy(cbuf.at[0, pl.ds(0, nz_tail)], SW.at[pl.ds(nb, nz_tail)])
    plsc.subcore_barrier()

    def issue(ci, sl2, sl4):
        pltpu.sync_copy(idx_hbm.at[ci], idxb.at[sl4])
        pltpu.async_copy(h_hbm.at[cid].at[idxb.at[sl4, 0]],
                         hrows.at[sl2], sem_h.at[sl2])
        pltpu.async_copy(emb_hbm.at[cid, pl.ds(ci * _C, _C)],
                         erows.at[sl2], sem_e.at[sl2])

    issue(cstart, 0, 0)
    issue(cstart + 1, 1, 1)

    def body(i, carry):
        p = lax.rem(i, 2)
        p4 = lax.rem(i, 4)
        ci = cstart + i
        pltpu.make_async_copy(h_hbm.at[cid].at[idxb.at[p4, 0]],
                              hrows.at[p], sem_h.at[p]).wait()
        pltpu.make_async_copy(emb_hbm.at[cid, pl.ds(ci * _C, _C)],
                              erows.at[p], sem_e.at[p]).wait()

        @pl.when(i >= 2)
        def _():
            pltpu.make_async_copy(cbuf.at[p], SW.at[idxb.at[p4, 1]],
                                  sem_s.at[p]).wait()

        def comp(e, carry2):
            for k in range(4):
                sl = pl.ds(k * 16, 16)
                msg = jnp.maximum(hrows[p, e, sl] + erows[p, e, sl], 0.0) + EPS
                ex = jnp.exp(msg)
                cbuf[p, e, sl] = ex
                cbuf[p, e, pl.ds(H2 + k * 16, 16)] = ex * msg
            return carry2

        lax.fori_loop(0, _C, comp, 0, unroll=False)
        pltpu.async_copy(cbuf.at[p], SW.at[idxb.at[p4, 1]], sem_s.at[p],
                         add=True)

        @pl.when(i + 2 < cnt)
        def _():
            issue(ci + 2, p, lax.rem(i + 2, 4))

        return carry

    lax.fori_loop(0, cnt, body, 0, unroll=False)
    for q in range(2):
        pltpu.make_async_copy(cbuf.at[q], SW.at[idxb.at[q, 1]],
                              sem_s.at[q]).wait()
    plsc.subcore_barrier()

    def fchunk(j, carry):
        nb = sid * _NPT + j * _FC
        pltpu.sync_copy(SW.at[pl.ds(nb, _FC)], cbuf.at[0, pl.ds(0, _FC)])

        def fbody(e, carry2):
            for k in range(4):
                s = cbuf[0, e, pl.ds(k * 16, 16)]
                w = cbuf[0, e, pl.ds(H2 + k * 16, 16)]
                hrows[0, e, pl.ds(k * 16, 16)] = w / (s + 1e-16)
            return carry2

        lax.fori_loop(0, _FC, fbody, 0, unroll=False)
        pltpu.sync_copy(hrows.at[0, pl.ds(0, _FC)],
                        out_hbm.at[cid, pl.ds(nb, _FC)])
        return carry

    lax.fori_loop(0, _NPT // _FC, fchunk, 0, unroll=False)


def _conv_sc(h_split, emb_split, idx_packed):
    kern = pl.kernel(
        _conv_body,
        out_type=jax.ShapeDtypeStruct((2, N, H2), jnp.float32),
        mesh=_mesh,
        scratch_types=[
            pltpu.VMEM((4, 2, _C), jnp.int32),
            pltpu.VMEM((2, _C, H2), jnp.float32),
            pltpu.VMEM((2, _C, H2), jnp.float32),
            pltpu.VMEM((2, _C, H), jnp.float32),
            pltpu.VMEM_SHARED((N, H), jnp.float32),
            pltpu.SemaphoreType.DMA((2,)),
            pltpu.SemaphoreType.DMA((2,)),
            pltpu.SemaphoreType.DMA((2,)),
        ],
        compiler_params=pltpu.CompilerParams(use_tc_tiling_on_sc=False),
    )
    return kern(h_split, emb_split, idx_packed)


def kernel(x, edge_index, edge_attr, W_node, b_node, W_edge, b_edge,
           Wc0, bc0, Wc1, bc1, g0, be0, g1, be1):
    idx_packed = edge_index.reshape(2, _NCH, _C).transpose(1, 0, 2)
    b_node = b_node.reshape(1, H)
    b_edge = b_edge.reshape(1, H)
    bc0 = bc0.reshape(1, H)
    bc1 = bc1.reshape(1, H)
    g0 = g0.reshape(1, H)
    g1 = g1.reshape(1, H)
    be0 = be0.reshape(1, H)
    be1 = be1.reshape(1, H)

    h0 = _enc_node(x, W_node, b_node)
    emb = _enc_edge(edge_attr, W_edge, b_edge)
    a1 = _conv_sc(h0, emb, idx_packed)
    h2 = _mlp(h0, a1, Wc0, bc0, g0, be0, relu_out=True)
    a2 = _conv_sc(h2, emb, idx_packed)
    return _mlp(h2, a2, Wc1, bc1, g1, be1, relu_out=False)
